# baseline (speedup 1.0000x reference)
import functools

import jax
import jax.numpy as jnp
from jax import lax
from jax.experimental import pallas as pl
from jax.experimental.pallas import tpu as pltpu
from jax.experimental.pallas import tpu_sc as plsc

D = 1024
H = 2048
E = 8
TK = 2

BM = 256
NT = 40
T = 4096
A = T * TK

NC, NS = 2, 16
NW = NC * NS


def _router_kernel(x_ref, gw_ref, i1_ref, i2_ref, g1_ref, g2_ref):
    x = x_ref[...]
    logits = jnp.dot(x, gw_ref[...], preferred_element_type=jnp.float32)
    i1 = jnp.argmax(logits, axis=-1)
    iota = lax.broadcasted_iota(jnp.int32, logits.shape, 1)
    masked = jnp.where(iota == i1[:, None], -jnp.inf, logits)
    i2 = jnp.argmax(masked, axis=-1)
    m1 = jnp.max(logits, axis=-1)
    m2 = jnp.max(masked, axis=-1)
    b = jnp.exp(m2 - m1)
    g1 = 1.0 / (1.0 + b)
    g2 = b / (1.0 + b)
    i1_ref[...] = i1.astype(jnp.int32)
    i2_ref[...] = i2.astype(jnp.int32)
    g1_ref[...] = jnp.broadcast_to(g1[:, None], g1_ref.shape)
    g2_ref[...] = jnp.broadcast_to(g2[:, None], g2_ref.shape)


def _router(xf, gate_w):
    bm = 2048
    return pl.pallas_call(
        _router_kernel,
        grid=(T // bm,),
        in_specs=[
            pl.BlockSpec((bm, D), lambda i: (i, 0)),
            pl.BlockSpec((D, E), lambda i: (0, 0)),
        ],
        out_specs=[
            pl.BlockSpec((bm,), lambda i: (i,)),
            pl.BlockSpec((bm,), lambda i: (i,)),
            pl.BlockSpec((bm, 16), lambda i: (i, 0)),
            pl.BlockSpec((bm, 16), lambda i: (i, 0)),
        ],
        out_shape=[
            jax.ShapeDtypeStruct((T,), jnp.int32),
            jax.ShapeDtypeStruct((T,), jnp.int32),
            jax.ShapeDtypeStruct((T, 16), jnp.float32),
            jax.ShapeDtypeStruct((T, 16), jnp.float32),
        ],
    )(xf, gate_w)


_DCH = 32


@functools.cache
def _make_sc_dispatch():
    mesh = plsc.VectorSubcoreMesh(core_axis_name="c", subcore_axis_name="s",
                                  num_cores=NC, num_subcores=NS)

    @functools.partial(
        pl.kernel,
        mesh=mesh,
        out_type=jax.ShapeDtypeStruct((NT * BM, D), jnp.float32),
        scratch_types=[
            pltpu.VMEM((_DCH,), jnp.int32),
            pltpu.VMEM((_DCH,), jnp.int32),
            pltpu.VMEM((_DCH, D), jnp.float32),
            pltpu.SemaphoreType.DMA,
        ],
    )
    def k(xf_hbm, p0_hbm, p1_hbm, out_hbm, p0_v, p1_v, rows_v, sem):
        wid = lax.axis_index("s") * NC + lax.axis_index("c")
        base = wid * (T // NW)
        for ch in range(T // NW // _DCH):
            off = base + ch * _DCH
            pltpu.sync_copy(xf_hbm.at[pl.ds(off, _DCH)], rows_v)
            pltpu.sync_copy(p0_hbm.at[pl.ds(off, _DCH)], p0_v)
            pltpu.sync_copy(p1_hbm.at[pl.ds(off, _DCH)], p1_v)
            pltpu.async_copy(rows_v, out_hbm.at[p0_v], sem).wait()
            pltpu.async_copy(rows_v, out_hbm.at[p1_v], sem).wait()

    return k


def _dispatch_rows(xf, pos0, pos1):
    return _make_sc_dispatch()(xf, pos0, pos1)


_CCH = 16


@functools.cache
def _make_sc_combine():
    mesh = plsc.VectorSubcoreMesh(core_axis_name="c", subcore_axis_name="s",
                                  num_cores=NC, num_subcores=NS)

    buf_set = [
        pltpu.VMEM((_CCH,), jnp.int32),
        pltpu.VMEM((_CCH,), jnp.int32),
        pltpu.VMEM((_CCH, 16), jnp.float32),
        pltpu.VMEM((_CCH, 16), jnp.float32),
        pltpu.VMEM((_CCH, D), jnp.float32),
        pltpu.VMEM((_CCH, D), jnp.float32),
        pltpu.SemaphoreType.DMA,
        pltpu.SemaphoreType.DMA,
    ]

    @functools.partial(
        pl.kernel,
        mesh=mesh,
        out_type=jax.ShapeDtypeStruct((T, D), jnp.float32),
        scratch_types=buf_set + buf_set,
    )
    def k(y_hbm, p0_hbm, p1_hbm, g0_hbm, g1_hbm, out_hbm, *scr):
        sets = (scr[:8], scr[8:])
        wid = lax.axis_index("s") * NC + lax.axis_index("c")
        base = wid * (T // NW)
        nch = T // NW // _CCH

        def start(st, off):
            p0_v, p1_v, w0_v, w1_v, b0_v, b1_v, s0, s1 = st
            pltpu.sync_copy(p0_hbm.at[pl.ds(off, _CCH)], p0_v)
            pltpu.sync_copy(p1_hbm.at[pl.ds(off, _CCH)], p1_v)
            pltpu.sync_copy(g0_hbm.at[pl.ds(off, _CCH)], w0_v)
            pltpu.sync_copy(g1_hbm.at[pl.ds(off, _CCH)], w1_v)
            cp0 = pltpu.async_copy(y_hbm.at[p0_v], b0_v, s0)
            cp1 = pltpu.async_copy(y_hbm.at[p1_v], b1_v, s1)
            return cp0, cp1

        pending = start(sets[0], base)
        for ch in range(nch):
            cur = sets[ch % 2]
            if ch + 1 < nch:
                nxt = start(sets[(ch + 1) % 2], base + (ch + 1) * _CCH)
            pending[0].wait()
            pending[1].wait()
            _, _, w0_v, w1_v, b0_v, b1_v, _, _ = cur

            def row_body(r, carry):
                s0v = w0_v[r, :]
                s1v = w1_v[r, :]
                for c in range(D // 16):
                    sl = pl.ds(c * 16, 16)
                    b0_v[r, sl] = b0_v[r, sl] * s0v + b1_v[r, sl] * s1v
                return carry

            lax.fori_loop(0, _CCH, row_body, 0)
            pltpu.sync_copy(b0_v, out_hbm.at[pl.ds(base + ch * _CCH, _CCH)])
            if ch + 1 < nch:
                pending = nxt

    return k


def _combine_rows(y_sorted, pos0, pos1, g0b, g1b):
    return _make_sc_combine()(y_sorted, pos0, pos1, g0b, g1b)


def _gmm_kernel(gid_ref, nreal_ref,
                x_ref, w1_ref, b1_ref, w2_ref, b2_ref,
                w3_ref, b3_ref, out_ref):
    j = pl.program_id(0)

    @pl.when(j < nreal_ref[0])
    def _run():
        x = x_ref[...]
        h1 = jnp.dot(x, w1_ref[0], preferred_element_type=jnp.float32)
        h2 = jnp.dot(x, w2_ref[0], preferred_element_type=jnp.float32)
        h1 = h1 + b1_ref[0]
        h2 = h2 + b2_ref[0]
        hh = h1 * (h2 * jax.nn.sigmoid(h2))
        out_ref[...] = jnp.dot(hh, w3_ref[0],
                               preferred_element_type=jnp.float32) + b3_ref[0]


def _gmm(gid, nreal, x_sorted, w1, b1r, w2, b2r, w3, b3r):
    grid_spec = pltpu.PrefetchScalarGridSpec(
        num_scalar_prefetch=2,
        grid=(NT,),
        in_specs=[
            pl.BlockSpec((BM, D), lambda j, g, n: (j, 0)),
            pl.BlockSpec((1, D, H), lambda j, g, n: (g[j], 0, 0)),
            pl.BlockSpec((1, 1, H), lambda j, g, n: (g[j], 0, 0)),
            pl.BlockSpec((1, D, H), lambda j, g, n: (g[j], 0, 0)),
            pl.BlockSpec((1, 1, H), lambda j, g, n: (g[j], 0, 0)),
            pl.BlockSpec((1, H, D), lambda j, g, n: (g[j], 0, 0)),
            pl.BlockSpec((1, 1, D), lambda j, g, n: (g[j], 0, 0)),
        ],
        out_specs=pl.BlockSpec((BM, D), lambda j, g, n: (j, 0)),
    )
    return pl.pallas_call(
        _gmm_kernel,
        grid_spec=grid_spec,
        out_shape=jax.ShapeDtypeStruct((NT * BM, D), jnp.float32),
        compiler_params=pltpu.CompilerParams(
            dimension_semantics=("arbitrary",),
            vmem_limit_bytes=100 * 1024 * 1024),
    )(gid, nreal, x_sorted, w1, b1r, w2, b2r, w3, b3r)


def _route_metadata(i1, i2):
    i32 = jnp.int32
    e_a = jnp.concatenate([i1, i2])
    onehot = (e_a[:, None] == jnp.arange(E, dtype=i32)[None, :]).astype(i32)
    inc = jnp.cumsum(onehot, axis=0)
    counts = inc[-1]
    gt = ((counts + BM - 1) // BM).astype(i32)
    tile_start = jnp.concatenate([jnp.zeros(1, i32),
                                  jnp.cumsum(gt).astype(i32)])
    aligned_off = tile_start[:E] * BM
    rank = jnp.sum((inc - onehot) * onehot, axis=1)
    base = jnp.sum(aligned_off[None, :] * onehot, axis=1)
    pos = base + rank
    pos0 = pos[:T]
    pos1 = pos[T:]
    n_real = tile_start[E]
    jarr = jnp.arange(NT, dtype=i32)
    gid = jnp.clip(
        jnp.searchsorted(tile_start, jarr, side="right").astype(i32) - 1,
        0, E - 1)
    return pos0, pos1, gid, n_real[None]


def kernel(x, gate_w, w1, b1, w2, b2, w3, b3):
    orig_shape = x.shape
    xf = x.reshape(-1, D)
    i1, i2, g1b, g2b = _router(xf, gate_w)
    pos0, pos1, gid, nreal = _route_metadata(i1, i2)

    x_sorted = _dispatch_rows(xf, pos0, pos1)

    y_sorted = _gmm(gid, nreal, x_sorted,
                    w1, b1.reshape(E, 1, H),
                    w2, b2.reshape(E, 1, H),
                    w3, b3.reshape(E, 1, D))

    out = _combine_rows(y_sorted, pos0, pos1, g1b, g2b)
    return out.reshape(orig_shape)


# speedup vs baseline: 1.0673x; 1.0673x over previous
import functools

import jax
import jax.numpy as jnp
from jax import lax
from jax.experimental import pallas as pl
from jax.experimental.pallas import tpu as pltpu
from jax.experimental.pallas import tpu_sc as plsc

D = 1024
H = 2048
E = 8
TK = 2

BM = 512
NT = 23
T = 4096
A = T * TK

NC, NS = 2, 16
NW = NC * NS


def _router_kernel(x_ref, gw_ref, i1_ref, i2_ref, g1_ref, g2_ref):
    x = x_ref[...]
    logits = jnp.dot(x, gw_ref[...], preferred_element_type=jnp.float32)
    i1 = jnp.argmax(logits, axis=-1)
    iota = lax.broadcasted_iota(jnp.int32, logits.shape, 1)
    masked = jnp.where(iota == i1[:, None], -jnp.inf, logits)
    i2 = jnp.argmax(masked, axis=-1)
    m1 = jnp.max(logits, axis=-1)
    m2 = jnp.max(masked, axis=-1)
    b = jnp.exp(m2 - m1)
    g1 = 1.0 / (1.0 + b)
    g2 = b / (1.0 + b)
    i1_ref[...] = i1.astype(jnp.int32)
    i2_ref[...] = i2.astype(jnp.int32)
    g1_ref[...] = jnp.broadcast_to(g1[:, None], g1_ref.shape)
    g2_ref[...] = jnp.broadcast_to(g2[:, None], g2_ref.shape)


def _router(xf, gate_w):
    bm = 2048
    return pl.pallas_call(
        _router_kernel,
        grid=(T // bm,),
        in_specs=[
            pl.BlockSpec((bm, D), lambda i: (i, 0)),
            pl.BlockSpec((D, E), lambda i: (0, 0)),
        ],
        out_specs=[
            pl.BlockSpec((bm,), lambda i: (i,)),
            pl.BlockSpec((bm,), lambda i: (i,)),
            pl.BlockSpec((bm, 16), lambda i: (i, 0)),
            pl.BlockSpec((bm, 16), lambda i: (i, 0)),
        ],
        out_shape=[
            jax.ShapeDtypeStruct((T,), jnp.int32),
            jax.ShapeDtypeStruct((T,), jnp.int32),
            jax.ShapeDtypeStruct((T, 16), jnp.float32),
            jax.ShapeDtypeStruct((T, 16), jnp.float32),
        ],
    )(xf, gate_w)


_DCH = 16


@functools.cache
def _make_sc_dispatch():
    mesh = plsc.VectorSubcoreMesh(core_axis_name="c", subcore_axis_name="s",
                                  num_cores=NC, num_subcores=NS)

    buf_set = [
        pltpu.VMEM((_DCH,), jnp.int32),
        pltpu.VMEM((_DCH,), jnp.int32),
        pltpu.VMEM((_DCH, D), jnp.float32),
        pltpu.SemaphoreType.DMA,
        pltpu.SemaphoreType.DMA,
        pltpu.SemaphoreType.DMA,
    ]

    @functools.partial(
        pl.kernel,
        mesh=mesh,
        out_type=jax.ShapeDtypeStruct((NT * BM, D), jnp.float32),
        scratch_types=buf_set + buf_set,
    )
    def k(xf_hbm, p0_hbm, p1_hbm, out_hbm, *scr):
        sets = (scr[:6], scr[6:])
        wid = lax.axis_index("s") * NC + lax.axis_index("c")
        base = wid * (T // NW)
        nch = T // NW // _DCH

        def start(st, off):
            p0_v, p1_v, rows_v, sr, _, _ = st
            pltpu.sync_copy(p0_hbm.at[pl.ds(off, _DCH)], p0_v)
            pltpu.sync_copy(p1_hbm.at[pl.ds(off, _DCH)], p1_v)
            return pltpu.async_copy(xf_hbm.at[pl.ds(off, _DCH)], rows_v, sr)

        pending = start(sets[0], base)
        scat = None
        for ch in range(nch):
            cur = sets[ch % 2]
            if scat is not None:
                scat[0].wait()
                scat[1].wait()
            if ch + 1 < nch:
                nxt_pending = start(sets[(ch + 1) % 2],
                                    base + (ch + 1) * _DCH)
            pending.wait()
            p0_v, p1_v, rows_v, _, s0, s1 = cur
            scat = (pltpu.async_copy(rows_v, out_hbm.at[p0_v], s0),
                    pltpu.async_copy(rows_v, out_hbm.at[p1_v], s1))
            if ch + 1 < nch:
                pending = nxt_pending
        scat[0].wait()
        scat[1].wait()

    return k


def _dispatch_rows(xf, pos0, pos1):
    return _make_sc_dispatch()(xf, pos0, pos1)


_CCH = 16


@functools.cache
def _make_sc_combine():
    mesh = plsc.VectorSubcoreMesh(core_axis_name="c", subcore_axis_name="s",
                                  num_cores=NC, num_subcores=NS)

    buf_set = [
        pltpu.VMEM((_CCH,), jnp.int32),
        pltpu.VMEM((_CCH,), jnp.int32),
        pltpu.VMEM((_CCH, 16), jnp.float32),
        pltpu.VMEM((_CCH, 16), jnp.float32),
        pltpu.VMEM((_CCH, D), jnp.float32),
        pltpu.VMEM((_CCH, D), jnp.float32),
        pltpu.SemaphoreType.DMA,
        pltpu.SemaphoreType.DMA,
    ]

    @functools.partial(
        pl.kernel,
        mesh=mesh,
        out_type=jax.ShapeDtypeStruct((T, D), jnp.float32),
        scratch_types=buf_set + buf_set,
    )
    def k(y_hbm, p0_hbm, p1_hbm, g0_hbm, g1_hbm, out_hbm, *scr):
        sets = (scr[:8], scr[8:])
        wid = lax.axis_index("s") * NC + lax.axis_index("c")
        base = wid * (T // NW)
        nch = T // NW // _CCH

        def start(st, off):
            p0_v, p1_v, w0_v, w1_v, b0_v, b1_v, s0, s1 = st
            pltpu.sync_copy(p0_hbm.at[pl.ds(off, _CCH)], p0_v)
            pltpu.sync_copy(p1_hbm.at[pl.ds(off, _CCH)], p1_v)
            pltpu.sync_copy(g0_hbm.at[pl.ds(off, _CCH)], w0_v)
            pltpu.sync_copy(g1_hbm.at[pl.ds(off, _CCH)], w1_v)
            cp0 = pltpu.async_copy(y_hbm.at[p0_v], b0_v, s0)
            cp1 = pltpu.async_copy(y_hbm.at[p1_v], b1_v, s1)
            return cp0, cp1

        pending = start(sets[0], base)
        for ch in range(nch):
            cur = sets[ch % 2]
            if ch + 1 < nch:
                nxt = start(sets[(ch + 1) % 2], base + (ch + 1) * _CCH)
            pending[0].wait()
            pending[1].wait()
            _, _, w0_v, w1_v, b0_v, b1_v, _, _ = cur

            def row_body(r, carry):
                s0v = w0_v[r, :]
                s1v = w1_v[r, :]
                for c in range(D // 16):
                    sl = pl.ds(c * 16, 16)
                    b0_v[r, sl] = b0_v[r, sl] * s0v + b1_v[r, sl] * s1v
                return carry

            lax.fori_loop(0, _CCH, row_body, 0)
            pltpu.sync_copy(b0_v, out_hbm.at[pl.ds(base + ch * _CCH, _CCH)])
            if ch + 1 < nch:
                pending = nxt

    return k


def _combine_rows(y_sorted, pos0, pos1, g0b, g1b):
    return _make_sc_combine()(y_sorted, pos0, pos1, g0b, g1b)


def _gmm_kernel(gid_ref, nreal_ref,
                x_ref, w1_ref, b1_ref, w2_ref, b2_ref,
                w3_ref, b3_ref, out_ref):
    j = pl.program_id(0)

    @pl.when(j < nreal_ref[0])
    def _run():
        x = x_ref[...]
        h1 = jnp.dot(x, w1_ref[0], preferred_element_type=jnp.float32)
        h2 = jnp.dot(x, w2_ref[0], preferred_element_type=jnp.float32)
        h1 = h1 + b1_ref[0]
        h2 = h2 + b2_ref[0]
        hh = h1 * (h2 * jax.nn.sigmoid(h2))
        out_ref[...] = jnp.dot(hh, w3_ref[0],
                               preferred_element_type=jnp.float32) + b3_ref[0]


def _gmm(gid, nreal, x_sorted, w1, b1r, w2, b2r, w3, b3r):
    grid_spec = pltpu.PrefetchScalarGridSpec(
        num_scalar_prefetch=2,
        grid=(NT,),
        in_specs=[
            pl.BlockSpec((BM, D), lambda j, g, n: (j, 0)),
            pl.BlockSpec((1, D, H), lambda j, g, n: (g[j], 0, 0)),
            pl.BlockSpec((1, 1, H), lambda j, g, n: (g[j], 0, 0)),
            pl.BlockSpec((1, D, H), lambda j, g, n: (g[j], 0, 0)),
            pl.BlockSpec((1, 1, H), lambda j, g, n: (g[j], 0, 0)),
            pl.BlockSpec((1, H, D), lambda j, g, n: (g[j], 0, 0)),
            pl.BlockSpec((1, 1, D), lambda j, g, n: (g[j], 0, 0)),
        ],
        out_specs=pl.BlockSpec((BM, D), lambda j, g, n: (j, 0)),
    )
    return pl.pallas_call(
        _gmm_kernel,
        grid_spec=grid_spec,
        out_shape=jax.ShapeDtypeStruct((NT * BM, D), jnp.float32),
        compiler_params=pltpu.CompilerParams(
            dimension_semantics=("arbitrary",),
            vmem_limit_bytes=100 * 1024 * 1024),
    )(gid, nreal, x_sorted, w1, b1r, w2, b2r, w3, b3r)


def _route_metadata(i1, i2):
    i32 = jnp.int32
    e_a = jnp.concatenate([i1, i2])
    onehot = (e_a[:, None] == jnp.arange(E, dtype=i32)[None, :]).astype(i32)
    inc = jnp.cumsum(onehot, axis=0)
    counts = inc[-1]
    gt = ((counts + BM - 1) // BM).astype(i32)
    tile_start = jnp.concatenate([jnp.zeros(1, i32),
                                  jnp.cumsum(gt).astype(i32)])
    aligned_off = tile_start[:E] * BM
    rank = jnp.sum((inc - onehot) * onehot, axis=1)
    base = jnp.sum(aligned_off[None, :] * onehot, axis=1)
    pos = base + rank
    pos0 = pos[:T]
    pos1 = pos[T:]
    n_real = tile_start[E]
    jarr = jnp.arange(NT, dtype=i32)
    gid = jnp.clip(
        jnp.searchsorted(tile_start, jarr, side="right").astype(i32) - 1,
        0, E - 1)
    return pos0, pos1, gid, n_real[None]


def kernel(x, gate_w, w1, b1, w2, b2, w3, b3):
    orig_shape = x.shape
    xf = x.reshape(-1, D)
    i1, i2, g1b, g2b = _router(xf, gate_w)
    pos0, pos1, gid, nreal = _route_metadata(i1, i2)

    x_sorted = _dispatch_rows(xf, pos0, pos1)

    y_sorted = _gmm(gid, nreal, x_sorted,
                    w1, b1.reshape(E, 1, H),
                    w2, b2.reshape(E, 1, H),
                    w3, b3.reshape(E, 1, D))

    out = _combine_rows(y_sorted, pos0, pos1, g1b, g2b)
    return out.reshape(orig_shape)
